# baseline (speedup 1.0000x reference)
import functools

import jax
import jax.numpy as jnp
from jax import lax
from jax.experimental import pallas as pl
from jax.experimental.pallas import tpu as pltpu
from jax.experimental.pallas import tpu_sc as plsc

D = 64
DP = 128
NC = 2
NS = 16
NW = NC * NS
B = 4096 * 200
ROWS_PER_W = B // NW
IDXW = 128
IDX_ROWS_PER_W = ROWS_PER_W // IDXW
CHUNK_IDX_ROWS = 4
CHUNK = CHUNK_IDX_ROWS * IDXW
N_CHUNKS = ROWS_PER_W // CHUNK
GROUPS = CHUNK // 16


def _emb_body(table_hbm, idx_hbm, out_hbm, idx_v, rows_a, rows_b,
              gsem_a, gsem_b, wsem_a, wsem_b):
    wid = lax.axis_index("s") * NC + lax.axis_index("c")
    pltpu.sync_copy(idx_hbm.at[wid], idx_v)
    base_w = wid * ROWS_PER_W

    def gather_copies(c, buf, sem):
        return [
            pltpu.make_async_copy(
                table_hbm.at[idx_v.at[c * CHUNK_IDX_ROWS + j]],
                buf.at[pl.ds(j * IDXW, IDXW)],
                sem,
            )
            for j in range(CHUNK_IDX_ROWS)
        ]

    def wb_copy(c, buf, sem):
        return pltpu.make_async_copy(
            buf,
            out_hbm.at[pl.ds(base_w + c * CHUNK, CHUNK), pl.ds(0, D)],
            sem,
        )

    def fixup(c, buf):
        def group_body(g, carry2):
            row = c * CHUNK_IDX_ROWS + g // (IDXW // 16)
            off = (g % (IDXW // 16)) * 16
            v = idx_v[row, pl.ds(off, 16)]
            m = v == 0
            nz = jnp.max(m.astype(jnp.int32))

            @pl.when(nz > 0)
            def _():
                rows0 = g * 16 + lax.iota(jnp.int32, 16)
                zeros = jnp.zeros((16,), jnp.float32)

                def col_body(col, carry3):
                    cols = jnp.full((16,), col, jnp.int32)
                    plsc.store_scatter(buf, [rows0, cols], zeros, mask=m)
                    return carry3

                lax.fori_loop(0, D, col_body, 0)

            return carry2

        lax.fori_loop(0, GROUPS, group_body, 0)

    for cp in gather_copies(0, rows_a, gsem_a):
        cp.start()

    def pair_body(i2, carry):
        c0 = 2 * i2
        c1 = c0 + 1

        for cp in gather_copies(c0, rows_a, gsem_a):
            cp.wait()

        @pl.when(i2 > 0)
        def _():
            wb_copy(c1 - 2, rows_b, wsem_b).wait()

        for cp in gather_copies(c1, rows_b, gsem_b):
            cp.start()

        fixup(c0, rows_a)
        wb_copy(c0, rows_a, wsem_a).start()

        for cp in gather_copies(c1, rows_b, gsem_b):
            cp.wait()

        @pl.when(i2 + 1 < N_CHUNKS // 2)
        def _():
            wb_copy(c0, rows_a, wsem_a).wait()
            for cp in gather_copies(c0 + 2, rows_a, gsem_a):
                cp.start()

        fixup(c1, rows_b)
        wb_copy(c1, rows_b, wsem_b).start()
        return carry

    lax.fori_loop(0, N_CHUNKS // 2, pair_body, 0)

    wb_copy(N_CHUNKS - 2, rows_a, wsem_a).wait()
    wb_copy(N_CHUNKS - 1, rows_b, wsem_b).wait()


_emb = functools.partial(
    pl.kernel,
    mesh=plsc.VectorSubcoreMesh(
        core_axis_name="c", subcore_axis_name="s", num_cores=NC
    ),
    compiler_params=pltpu.CompilerParams(
        use_tc_tiling_on_sc=False, needs_layout_passes=False
    ),
    out_type=jax.ShapeDtypeStruct((B, DP), jnp.float32),
    scratch_types=[
        pltpu.VMEM((IDX_ROWS_PER_W, IDXW), jnp.int32),
        pltpu.VMEM((CHUNK, D), jnp.float32),
        pltpu.VMEM((CHUNK, D), jnp.float32),
        pltpu.SemaphoreType.DMA,
        pltpu.SemaphoreType.DMA,
        pltpu.SemaphoreType.DMA,
        pltpu.SemaphoreType.DMA,
    ],
)(_emb_body)


def kernel(input_batch, table):
    bsz, seq = input_batch.shape
    tbl = jnp.stack([table, jnp.zeros_like(table)], axis=1)
    tbl = tbl.reshape(2 * 1000000, D)
    idx = input_batch.reshape(-1).astype(jnp.int32) * 2
    idx = idx.reshape(NW, IDX_ROWS_PER_W, IDXW)
    out = _emb(tbl, idx)
    return out.reshape(bsz, seq, DP)[:, :, :D]


# speedup vs baseline: 2.1051x; 2.1051x over previous
import functools

import jax
import jax.numpy as jnp
from jax import lax
from jax.experimental import pallas as pl
from jax.experimental.pallas import tpu as pltpu
from jax.experimental.pallas import tpu_sc as plsc

D = 64
DP = 128
NC = 2
NS = 16
NW = NC * NS
B = 4096 * 200
ROWS_PER_W = B // NW
IDXW = 128
IDX_ROWS_PER_W = ROWS_PER_W // IDXW
CHUNK_IDX_ROWS = 5
CHUNK = CHUNK_IDX_ROWS * IDXW
N_CHUNKS = ROWS_PER_W // CHUNK
GROUPS = CHUNK // 16


def _emb_body(table_hbm, idx_hbm, out_hbm, idx_v, rows_a, rows_b,
              gsem_a, gsem_b, wsem_a, wsem_b):
    wid = lax.axis_index("s") * NC + lax.axis_index("c")
    pltpu.sync_copy(idx_hbm.at[wid], idx_v)
    base_w = wid * ROWS_PER_W

    def gather_copies(c, buf, sem):
        return [
            pltpu.make_async_copy(
                table_hbm.at[idx_v.at[c * CHUNK_IDX_ROWS + j]],
                buf.at[pl.ds(j * IDXW, IDXW)],
                sem,
            )
            for j in range(CHUNK_IDX_ROWS)
        ]

    def wb_copy(c, buf, sem):
        return pltpu.make_async_copy(
            buf,
            out_hbm.at[pl.ds(base_w + c * CHUNK, CHUNK), pl.ds(0, D)],
            sem,
        )

    def fixup(c, buf):
        def group_body(g, carry2):
            row = c * CHUNK_IDX_ROWS + g // (IDXW // 16)
            off = (g % (IDXW // 16)) * 16
            v = idx_v[row, pl.ds(off, 16)]
            m = v == 0
            nz = jnp.max(m.astype(jnp.int32))

            @pl.when(nz > 0)
            def _():
                rows0 = g * 16 + lax.iota(jnp.int32, 16)
                zeros = jnp.zeros((16,), jnp.float32)

                def col_body(col, carry3):
                    cols = jnp.full((16,), col, jnp.int32)
                    plsc.store_scatter(buf, [rows0, cols], zeros, mask=m)
                    return carry3

                lax.fori_loop(0, D, col_body, 0)

            return carry2

        lax.fori_loop(0, GROUPS, group_body, 0)

    for cp in gather_copies(0, rows_a, gsem_a):
        cp.start()

    def pair_body(i2, carry):
        c0 = 2 * i2
        c1 = c0 + 1

        for cp in gather_copies(c0, rows_a, gsem_a):
            cp.wait()

        @pl.when(i2 > 0)
        def _():
            wb_copy(c1 - 2, rows_b, wsem_b).wait()

        for cp in gather_copies(c1, rows_b, gsem_b):
            cp.start()

        fixup(c0, rows_a)
        wb_copy(c0, rows_a, wsem_a).start()

        for cp in gather_copies(c1, rows_b, gsem_b):
            cp.wait()

        @pl.when(i2 + 1 < N_CHUNKS // 2)
        def _():
            wb_copy(c0, rows_a, wsem_a).wait()
            for cp in gather_copies(c0 + 2, rows_a, gsem_a):
                cp.start()

        fixup(c1, rows_b)
        wb_copy(c1, rows_b, wsem_b).start()
        return carry

    lax.fori_loop(0, N_CHUNKS // 2, pair_body, 0)

    wb_copy(N_CHUNKS - 2, rows_a, wsem_a).wait()
    wb_copy(N_CHUNKS - 1, rows_b, wsem_b).wait()


_emb = functools.partial(
    pl.kernel,
    mesh=plsc.VectorSubcoreMesh(
        core_axis_name="c", subcore_axis_name="s", num_cores=NC
    ),
    compiler_params=pltpu.CompilerParams(
        use_tc_tiling_on_sc=False, needs_layout_passes=False
    ),
    out_type=jax.ShapeDtypeStruct((B, DP), jnp.float32),
    scratch_types=[
        pltpu.VMEM((IDX_ROWS_PER_W, IDXW), jnp.int32),
        pltpu.VMEM((CHUNK, D), jnp.float32),
        pltpu.VMEM((CHUNK, D), jnp.float32),
        pltpu.SemaphoreType.DMA,
        pltpu.SemaphoreType.DMA,
        pltpu.SemaphoreType.DMA,
        pltpu.SemaphoreType.DMA,
    ],
)(_emb_body)


def kernel(input_batch, table):
    bsz, seq = input_batch.shape
    tbl = jnp.pad(table, ((0, 0), (0, DP - D))).reshape(2 * 1000000, D)
    idx = input_batch.reshape(-1).astype(jnp.int32) * 2
    idx = idx.reshape(NW, IDX_ROWS_PER_W, IDXW)
    out = _emb(tbl, idx)
    return out.reshape(bsz, seq, DP)[:, :, :D]
